# baseline (speedup 1.0000x reference)
import jax
import jax.numpy as jnp
from jax.experimental import pallas as pl
from jax.experimental.pallas import tpu as pltpu

T = 2048
HID = 2048
H = 16
NOPE = 128
ROPE = 64
VD = 128
QLORA = 1536
KVLORA = 512
IH = 16
IDIM = 128
TOPK = 512
SCALE = (NOPE + ROPE) ** -0.5
NEG = jnp.finfo(jnp.float32).min

TB = 256
QB = 256
NBLK = T // TB


def _bdot(a, b, dn=None):
    a = a.astype(jnp.bfloat16)
    b = b.astype(jnp.bfloat16)
    if dn is None:
        dn = (((a.ndim - 1,), (0,)), ((), ()))
    return jax.lax.dot_general(a, b, dn, preferred_element_type=jnp.float32)


def _rms(x, g, eps=1e-6):
    return x * jax.lax.rsqrt(jnp.mean(x * x, axis=-1, keepdims=True) + eps) * g


def _ln(x, g, b, eps=1e-6):
    m = jnp.mean(x, axis=-1, keepdims=True)
    v = jnp.mean((x - m) ** 2, axis=-1, keepdims=True)
    return (x - m) * jax.lax.rsqrt(v + eps) * g + b


def _rope2(x, cos, sin):
    half = ROPE // 2
    rot = jnp.concatenate([-x[:, half:], x[:, :half]], axis=1)
    return x * cos + rot * sin


def _rope3(x, cos, sin):
    half = ROPE // 2
    rot = jnp.concatenate([-x[:, :, half:], x[:, :, :half]], axis=2)
    return x * cos[:, None, :] + rot * sin[:, None, :]


def _prep_body(hs_ref, cos_ref, sin_ref, Wqa_ref, gqa_ref, Wkva_ref, gkva_ref,
               Wkvb_ref, Wik_ref, gik_ref, bik_ref, Wwt_ref, bwt_ref,
               qr_ref, knope_ref, v_ref, kpe_ref, ik_ref, w_ref):
    hs = hs_ref[...]
    cos = cos_ref[...]
    sin = sin_ref[...]
    qa = _bdot(hs, Wqa_ref[...])
    qr_ref[...] = _rms(qa, gqa_ref[...])
    kva = _bdot(hs, Wkva_ref[...])
    kv = _rms(kva[:, :KVLORA], gkva_ref[...])
    kpe_ref[...] = _rope2(kva[:, KVLORA:], cos, sin)
    kvb = _bdot(kv, Wkvb_ref[...])
    kvb3 = kvb.reshape(TB, H, NOPE + VD)
    knope_ref[...] = kvb3[:, :, :NOPE].reshape(TB, H * NOPE)
    v_ref[...] = kvb3[:, :, NOPE:].reshape(TB, H * VD)
    ik = _ln(_bdot(hs, Wik_ref[...]),
             gik_ref[...], bik_ref[...])
    ik_ref[...] = jnp.concatenate([_rope2(ik[:, :ROPE], cos, sin), ik[:, ROPE:]], axis=1)
    w_ref[...] = _bdot(hs, Wwt_ref[...]) + bwt_ref[...]


def _qproj_body(qr_ref, cos_ref, sin_ref, Wqb_ref, Wiq_ref,
                qnope_ref, qpe_ref, iq_ref):
    qr = qr_ref[...]
    cos = cos_ref[...]
    sin = sin_ref[...]
    q3 = _bdot(qr, Wqb_ref[...]).reshape(TB, H, NOPE + ROPE)
    qnope_ref[...] = q3[:, :, :NOPE].reshape(TB, H * NOPE)
    qpe_ref[...] = _rope3(q3[:, :, NOPE:], cos, sin).reshape(TB, H * ROPE)
    iq3 = _bdot(qr, Wiq_ref[...]).reshape(TB, IH, IDIM)
    iq3 = jnp.concatenate([_rope3(iq3[:, :, :ROPE], cos, sin), iq3[:, :, ROPE:]], axis=2)
    iq_ref[...] = iq3.reshape(TB, IH * IDIM)


def _make_main_body(kext, noff):
    def _main_body(qnope_ref, qpe_ref, iq_ref, w_ref, ik_ref, kpe_ref,
                   knope_ref, v_ref, out_ref):
        qb = noff + pl.program_id(0)
        ik = ik_ref[...]
        w = w_ref[...]

        scores = jnp.zeros((QB, kext), jnp.float32)
        for h in range(IH):
            iqh = iq_ref[:, h * IDIM:(h + 1) * IDIM]
            sh = _bdot(iqh, ik, (((1,), (1,)), ((), ())))
            shb = jnp.maximum(sh, 0.0).astype(jnp.bfloat16).astype(jnp.float32)
            wb = w[:, h][:, None].astype(jnp.bfloat16).astype(jnp.float32)
            scores = scores + wb * shb

        row = qb * QB + jax.lax.broadcasted_iota(jnp.int32, (QB, 1), 0)
        col = jax.lax.broadcasted_iota(jnp.int32, (QB, kext), 1)
        causal = col <= row
        scores = jnp.where(causal, scores, NEG)

        bits = jax.lax.bitcast_convert_type(scores, jnp.uint32)
        bits = jnp.where(bits == jnp.uint32(0x80000000), jnp.uint32(0), bits)
        keys = jnp.where(bits >= jnp.uint32(0x80000000), ~bits,
                         bits | jnp.uint32(0x80000000))
        ones = jnp.ones((kext, 8), jnp.float32)
        thr = jnp.zeros((QB, 1), jnp.uint32)
        for b in range(31, -1, -1):
            cand = thr | jnp.uint32(1 << b)
            gem = jnp.where(keys >= cand, 1.0, 0.0)
            cnt = _bdot(gem, ones)[:, :1]
            thr = jnp.where(cnt >= float(TOPK), cand, thr)
        mask = (keys >= thr) & causal

        kpe = kpe_ref[...]
        attn = []
        for h in range(H):
            qn = qnope_ref[:, h * NOPE:(h + 1) * NOPE]
            kn = knope_ref[:, h * NOPE:(h + 1) * NOPE]
            logits = _bdot(qn, kn, (((1,), (1,)), ((), ())))
            qp = qpe_ref[:, h * ROPE:(h + 1) * ROPE]
            logits = logits + _bdot(qp, kpe, (((1,), (1,)), ((), ())))
            logits = jnp.where(mask, logits * SCALE, NEG)
            m = jnp.max(logits, axis=1, keepdims=True)
            p = jnp.exp(logits - m)
            probs = p / jnp.sum(p, axis=1, keepdims=True)
            attn.append(_bdot(probs, v_ref[:, h * VD:(h + 1) * VD]))

        out_ref[...] = jnp.concatenate(attn, axis=1)

    return _main_body


def _oproj_body(x_ref, Wo_ref, out_ref):
    out_ref[...] = _bdot(x_ref[...], Wo_ref[...])


def _blk(shape, idx_fn):
    return pl.BlockSpec(shape, idx_fn)


_ROWB = lambda n: _blk((TB, n), lambda i: (i, 0))
_FULL = lambda m, n: _blk((m, n), lambda i: (0, 0))


def kernel(hidden_states, cos, sin, Wqa, g_qa, Wqb, Wkva, g_kva, Wkvb, Wo,
           Wiq, Wik, g_ik, b_ik, Wwt, bwt):
    f32 = jnp.float32
    gqa2 = g_qa.reshape(1, QLORA)
    gkva2 = g_kva.reshape(1, KVLORA)
    gik2 = g_ik.reshape(1, IDIM)
    bik2 = b_ik.reshape(1, IDIM)
    bwt2 = bwt.reshape(1, IH)

    params = pltpu.CompilerParams(vmem_limit_bytes=64 * 1024 * 1024)

    qr, knope, v, kpe, ik, w = pl.pallas_call(
        _prep_body,
        grid=(NBLK,),
        in_specs=[
            _ROWB(HID), _ROWB(ROPE), _ROWB(ROPE),
            _FULL(HID, QLORA), _FULL(1, QLORA),
            _FULL(HID, KVLORA + ROPE), _FULL(1, KVLORA),
            _FULL(KVLORA, H * (NOPE + VD)),
            _FULL(HID, IDIM), _FULL(1, IDIM), _FULL(1, IDIM),
            _FULL(HID, IH), _FULL(1, IH),
        ],
        out_specs=[
            _ROWB(QLORA), _ROWB(H * NOPE), _ROWB(H * VD),
            _ROWB(ROPE), _ROWB(IDIM), _ROWB(IH),
        ],
        out_shape=[
            jax.ShapeDtypeStruct((T, QLORA), f32),
            jax.ShapeDtypeStruct((T, H * NOPE), f32),
            jax.ShapeDtypeStruct((T, H * VD), f32),
            jax.ShapeDtypeStruct((T, ROPE), f32),
            jax.ShapeDtypeStruct((T, IDIM), f32),
            jax.ShapeDtypeStruct((T, IH), f32),
        ],
        compiler_params=params,
    )(hidden_states, cos, sin, Wqa, gqa2, Wkva, gkva2, Wkvb, Wik, gik2, bik2,
      Wwt, bwt2)

    qnope, qpe, iq = pl.pallas_call(
        _qproj_body,
        grid=(NBLK,),
        in_specs=[
            _ROWB(QLORA), _ROWB(ROPE), _ROWB(ROPE),
            _FULL(QLORA, H * (NOPE + ROPE)), _FULL(QLORA, IH * IDIM),
        ],
        out_specs=[_ROWB(H * NOPE), _ROWB(H * ROPE), _ROWB(IH * IDIM)],
        out_shape=[
            jax.ShapeDtypeStruct((T, H * NOPE), f32),
            jax.ShapeDtypeStruct((T, H * ROPE), f32),
            jax.ShapeDtypeStruct((T, IH * IDIM), f32),
        ],
        compiler_params=params,
    )(qr, cos, sin, Wqb, Wiq)

    GRP = 2
    attn_parts = []
    for g in range(T // QB // GRP):
        noff = g * GRP
        kext = (noff + GRP) * QB
        lo = noff * QB
        hi = lo + GRP * QB
        part = pl.pallas_call(
            _make_main_body(kext, noff),
            grid=(GRP,),
            in_specs=[
                _blk((QB, H * NOPE), lambda i: (i, 0)),
                _blk((QB, H * ROPE), lambda i: (i, 0)),
                _blk((QB, IH * IDIM), lambda i: (i, 0)),
                _blk((QB, IH), lambda i: (i, 0)),
                _blk((kext, IDIM), lambda i: (0, 0)),
                _blk((kext, ROPE), lambda i: (0, 0)),
                _blk((kext, H * NOPE), lambda i: (0, 0)),
                _blk((kext, H * VD), lambda i: (0, 0)),
            ],
            out_specs=_blk((QB, H * VD), lambda i: (i, 0)),
            out_shape=jax.ShapeDtypeStruct((GRP * QB, H * VD), f32),
            compiler_params=params,
        )(qnope[lo:hi], qpe[lo:hi], iq[lo:hi], w[lo:hi], ik, kpe, knope, v)
        attn_parts.append(part)
    attn = jnp.concatenate(attn_parts, axis=0)

    return pl.pallas_call(
        _oproj_body,
        grid=(NBLK,),
        in_specs=[_ROWB(H * VD), _FULL(H * VD, HID)],
        out_specs=_ROWB(HID),
        out_shape=jax.ShapeDtypeStruct((T, HID), f32),
        compiler_params=params,
    )(attn, Wo)


# speedup vs baseline: 1.1104x; 1.1104x over previous
import jax
import jax.numpy as jnp
from jax.experimental import pallas as pl
from jax.experimental.pallas import tpu as pltpu

T = 2048
HID = 2048
H = 16
NOPE = 128
ROPE = 64
VD = 128
QLORA = 1536
KVLORA = 512
IH = 16
IDIM = 128
TOPK = 512
SCALE = (NOPE + ROPE) ** -0.5
NEG = jnp.finfo(jnp.float32).min

TB = 256
QB = 256
NBLK = T // TB


def _bdot(a, b, dn=None):
    a = a.astype(jnp.bfloat16)
    b = b.astype(jnp.bfloat16)
    if dn is None:
        dn = (((a.ndim - 1,), (0,)), ((), ()))
    return jax.lax.dot_general(a, b, dn, preferred_element_type=jnp.float32)


def _rms(x, g, eps=1e-6):
    return x * jax.lax.rsqrt(jnp.mean(x * x, axis=-1, keepdims=True) + eps) * g


def _ln(x, g, b, eps=1e-6):
    m = jnp.mean(x, axis=-1, keepdims=True)
    v = jnp.mean((x - m) ** 2, axis=-1, keepdims=True)
    return (x - m) * jax.lax.rsqrt(v + eps) * g + b


def _rope2(x, cos, sin):
    half = ROPE // 2
    rot = jnp.concatenate([-x[:, half:], x[:, :half]], axis=1)
    return x * cos + rot * sin


def _rope3(x, cos, sin):
    half = ROPE // 2
    rot = jnp.concatenate([-x[:, :, half:], x[:, :, :half]], axis=2)
    return x * cos[:, None, :] + rot * sin[:, None, :]


def _prep_body(hs_ref, cos_ref, sin_ref, Wqa_ref, gqa_ref, Wkva_ref, gkva_ref,
               Wkvb_ref, Wik_ref, gik_ref, bik_ref, Wwt_ref, bwt_ref,
               qr_ref, knope_ref, v_ref, kpe_ref, ik_ref, w_ref):
    hs = hs_ref[...]
    cos = cos_ref[...]
    sin = sin_ref[...]
    qa = _bdot(hs, Wqa_ref[...])
    qr_ref[...] = _rms(qa, gqa_ref[...])
    kva = _bdot(hs, Wkva_ref[...])
    kv = _rms(kva[:, :KVLORA], gkva_ref[...])
    kpe_ref[...] = _rope2(kva[:, KVLORA:], cos, sin)
    kvb = _bdot(kv, Wkvb_ref[...])
    kvb3 = kvb.reshape(TB, H, NOPE + VD)
    knope_ref[...] = kvb3[:, :, :NOPE].reshape(TB, H * NOPE)
    v_ref[...] = kvb3[:, :, NOPE:].reshape(TB, H * VD)
    ik = _ln(_bdot(hs, Wik_ref[...]),
             gik_ref[...], bik_ref[...])
    ik_ref[...] = jnp.concatenate([_rope2(ik[:, :ROPE], cos, sin), ik[:, ROPE:]], axis=1)
    w_ref[...] = _bdot(hs, Wwt_ref[...]) + bwt_ref[...]


def _qproj_body(qr_ref, cos_ref, sin_ref, Wqb_ref, Wiq_ref,
                qnope_ref, qpe_ref, iq_ref):
    qr = qr_ref[...]
    cos = cos_ref[...]
    sin = sin_ref[...]
    q3 = _bdot(qr, Wqb_ref[...]).reshape(TB, H, NOPE + ROPE)
    qnope_ref[...] = q3[:, :, :NOPE].reshape(TB, H * NOPE)
    qpe_ref[...] = _rope3(q3[:, :, NOPE:], cos, sin).reshape(TB, H * ROPE)
    iq3 = _bdot(qr, Wiq_ref[...]).reshape(TB, IH, IDIM)
    iq3 = jnp.concatenate([_rope3(iq3[:, :, :ROPE], cos, sin), iq3[:, :, ROPE:]], axis=2)
    iq_ref[...] = iq3.reshape(TB, IH * IDIM)


def _make_main_body(kext, noff):
    def _main_body(qnope_ref, qpe_ref, iq_ref, w_ref, ik_ref, kpe_ref,
                   knope_ref, v_ref, out_ref):
        qb = noff + pl.program_id(0)
        ik = ik_ref[...]
        w = w_ref[...]

        scores = jnp.zeros((QB, kext), jnp.float32)
        for h in range(IH):
            iqh = iq_ref[:, h * IDIM:(h + 1) * IDIM]
            sh = _bdot(iqh, ik, (((1,), (1,)), ((), ())))
            shb = jnp.maximum(sh, 0.0).astype(jnp.bfloat16).astype(jnp.float32)
            wb = w[:, h][:, None].astype(jnp.bfloat16).astype(jnp.float32)
            scores = scores + wb * shb

        row = qb * QB + jax.lax.broadcasted_iota(jnp.int32, (QB, 1), 0)
        col = jax.lax.broadcasted_iota(jnp.int32, (QB, kext), 1)
        causal = col <= row
        scores = jnp.where(causal, scores, NEG)

        bits = jax.lax.bitcast_convert_type(scores, jnp.uint32)
        bits = jnp.where(bits == jnp.uint32(0x80000000), jnp.uint32(0), bits)
        keys = jnp.where(bits >= jnp.uint32(0x80000000), ~bits,
                         bits | jnp.uint32(0x80000000))
        thr = jnp.zeros((QB, 1), jnp.uint32)
        for b in range(31, -1, -1):
            cand = thr | jnp.uint32(1 << b)
            cnt = jnp.sum((keys >= cand).astype(jnp.int32), axis=1, keepdims=True)
            thr = jnp.where(cnt >= TOPK, cand, thr)
        mask = (keys >= thr) & causal

        kpe = kpe_ref[...]
        attn = []
        for h in range(H):
            qn = qnope_ref[:, h * NOPE:(h + 1) * NOPE]
            kn = knope_ref[:, h * NOPE:(h + 1) * NOPE]
            logits = _bdot(qn, kn, (((1,), (1,)), ((), ())))
            qp = qpe_ref[:, h * ROPE:(h + 1) * ROPE]
            logits = logits + _bdot(qp, kpe, (((1,), (1,)), ((), ())))
            logits = jnp.where(mask, logits * SCALE, NEG)
            m = jnp.max(logits, axis=1, keepdims=True)
            p = jnp.exp(logits - m)
            probs = p / jnp.sum(p, axis=1, keepdims=True)
            attn.append(_bdot(probs, v_ref[:, h * VD:(h + 1) * VD]))

        out_ref[...] = jnp.concatenate(attn, axis=1)

    return _main_body


def _oproj_body(x_ref, Wo_ref, out_ref):
    out_ref[...] = _bdot(x_ref[...], Wo_ref[...])


def _blk(shape, idx_fn):
    return pl.BlockSpec(shape, idx_fn)


_ROWB = lambda n: _blk((TB, n), lambda i: (i, 0))
_FULL = lambda m, n: _blk((m, n), lambda i: (0, 0))


def kernel(hidden_states, cos, sin, Wqa, g_qa, Wqb, Wkva, g_kva, Wkvb, Wo,
           Wiq, Wik, g_ik, b_ik, Wwt, bwt):
    f32 = jnp.float32
    gqa2 = g_qa.reshape(1, QLORA)
    gkva2 = g_kva.reshape(1, KVLORA)
    gik2 = g_ik.reshape(1, IDIM)
    bik2 = b_ik.reshape(1, IDIM)
    bwt2 = bwt.reshape(1, IH)

    params = pltpu.CompilerParams(vmem_limit_bytes=64 * 1024 * 1024)

    qr, knope, v, kpe, ik, w = pl.pallas_call(
        _prep_body,
        grid=(NBLK,),
        in_specs=[
            _ROWB(HID), _ROWB(ROPE), _ROWB(ROPE),
            _FULL(HID, QLORA), _FULL(1, QLORA),
            _FULL(HID, KVLORA + ROPE), _FULL(1, KVLORA),
            _FULL(KVLORA, H * (NOPE + VD)),
            _FULL(HID, IDIM), _FULL(1, IDIM), _FULL(1, IDIM),
            _FULL(HID, IH), _FULL(1, IH),
        ],
        out_specs=[
            _ROWB(QLORA), _ROWB(H * NOPE), _ROWB(H * VD),
            _ROWB(ROPE), _ROWB(IDIM), _ROWB(IH),
        ],
        out_shape=[
            jax.ShapeDtypeStruct((T, QLORA), f32),
            jax.ShapeDtypeStruct((T, H * NOPE), f32),
            jax.ShapeDtypeStruct((T, H * VD), f32),
            jax.ShapeDtypeStruct((T, ROPE), f32),
            jax.ShapeDtypeStruct((T, IDIM), f32),
            jax.ShapeDtypeStruct((T, IH), f32),
        ],
        compiler_params=params,
    )(hidden_states, cos, sin, Wqa, gqa2, Wkva, gkva2, Wkvb, Wik, gik2, bik2,
      Wwt, bwt2)

    qnope, qpe, iq = pl.pallas_call(
        _qproj_body,
        grid=(NBLK,),
        in_specs=[
            _ROWB(QLORA), _ROWB(ROPE), _ROWB(ROPE),
            _FULL(QLORA, H * (NOPE + ROPE)), _FULL(QLORA, IH * IDIM),
        ],
        out_specs=[_ROWB(H * NOPE), _ROWB(H * ROPE), _ROWB(IH * IDIM)],
        out_shape=[
            jax.ShapeDtypeStruct((T, H * NOPE), f32),
            jax.ShapeDtypeStruct((T, H * ROPE), f32),
            jax.ShapeDtypeStruct((T, IH * IDIM), f32),
        ],
        compiler_params=params,
    )(qr, cos, sin, Wqb, Wiq)

    GRP = 2
    attn_parts = []
    for g in range(T // QB // GRP):
        noff = g * GRP
        kext = (noff + GRP) * QB
        lo = noff * QB
        hi = lo + GRP * QB
        part = pl.pallas_call(
            _make_main_body(kext, noff),
            grid=(GRP,),
            in_specs=[
                _blk((QB, H * NOPE), lambda i: (i, 0)),
                _blk((QB, H * ROPE), lambda i: (i, 0)),
                _blk((QB, IH * IDIM), lambda i: (i, 0)),
                _blk((QB, IH), lambda i: (i, 0)),
                _blk((kext, IDIM), lambda i: (0, 0)),
                _blk((kext, ROPE), lambda i: (0, 0)),
                _blk((kext, H * NOPE), lambda i: (0, 0)),
                _blk((kext, H * VD), lambda i: (0, 0)),
            ],
            out_specs=_blk((QB, H * VD), lambda i: (i, 0)),
            out_shape=jax.ShapeDtypeStruct((GRP * QB, H * VD), f32),
            compiler_params=params,
        )(qnope[lo:hi], qpe[lo:hi], iq[lo:hi], w[lo:hi], ik, kpe, knope, v)
        attn_parts.append(part)
    attn = jnp.concatenate(attn_parts, axis=0)

    return pl.pallas_call(
        _oproj_body,
        grid=(NBLK,),
        in_specs=[_ROWB(H * VD), _FULL(H * VD, HID)],
        out_specs=_ROWB(HID),
        out_shape=jax.ShapeDtypeStruct((T, HID), f32),
        compiler_params=params,
    )(attn, Wo)
